# Initial kernel scaffold; baseline (speedup 1.0000x reference)
import jax
import jax.numpy as jnp
from jax.experimental import pallas as pl


def kernel(sample, ent_emb, rel_emb):
    raise NotImplementedError("write your pallas kernel here")


# speedup vs baseline: 1.2752x; 1.2752x over previous
import functools

import jax
import jax.numpy as jnp
from jax import lax
from jax.experimental import pallas as pl
from jax.experimental.pallas import tpu as pltpu
from jax.experimental.pallas import tpu_sc as plsc

_HID = 128
_GAMMA = 12.0
_ERANGE = (12.0 + 2.0) / _HID
_PI = 3.141592653589793
_PHASE_SCALE = _PI / _ERANGE

_B = 4096
_NW = 32
_BPW = _B // _NW
_LANES = 16


def _cs_body(rel_ref, cs_ref):
    ph = rel_ref[...] * _PHASE_SCALE
    cs_ref[:, :_HID] = jnp.cos(ph)
    cs_ref[:, _HID:] = jnp.sin(ph)


def _make_cs_table(rel_emb):
    n = rel_emb.shape[0]
    return pl.pallas_call(
        _cs_body,
        out_shape=jax.ShapeDtypeStruct((n, 2 * _HID), jnp.float32),
    )(rel_emb)


def _sc_score(ent_hbm, cs_hbm, hidx_hbm, ridx_hbm, tidx_hbm, out_hbm,
              ih, ir, it, hv, tv, cv, pv, ov, s1, s2, s3):
    wid = lax.axis_index("s") * 2 + lax.axis_index("c")
    base = wid * _BPW
    pltpu.sync_copy(hidx_hbm.at[pl.ds(base, _BPW)], ih)
    pltpu.sync_copy(ridx_hbm.at[pl.ds(base, _BPW)], ir)
    pltpu.sync_copy(tidx_hbm.at[pl.ds(base, _BPW)], it)
    c1 = pltpu.async_copy(ent_hbm.at[ih], hv, s1)
    c2 = pltpu.async_copy(ent_hbm.at[it], tv, s2)
    c3 = pltpu.async_copy(cs_hbm.at[ir], cv, s3)
    c1.wait()
    c2.wait()
    c3.wait()

    def body(i, carry):
        acc = jnp.zeros((_LANES,), jnp.float32)
        for c in range(_HID // _LANES):
            lo = c * _LANES
            reh = hv[i, pl.ds(lo, _LANES)]
            imh = hv[i, pl.ds(_HID + lo, _LANES)]
            ret = tv[i, pl.ds(lo, _LANES)]
            imt = tv[i, pl.ds(_HID + lo, _LANES)]
            cr = cv[i, pl.ds(lo, _LANES)]
            sr = cv[i, pl.ds(_HID + lo, _LANES)]
            re = reh * cr - imh * sr - ret
            im = reh * sr + imh * cr - imt
            s = re * re + im * im
            bits = lax.bitcast_convert_type(s, jnp.int32)
            r = lax.bitcast_convert_type(
                jnp.int32(0x5F3759DF) - (bits >> 1), jnp.float32)
            sh = 0.5 * s
            r = r * (1.5 - sh * r * r)
            r = r * (1.5 - sh * r * r)
            r = r * (1.5 - sh * r * r)
            acc = acc + s * r
        pv[i, pl.ds(0, _LANES)] = acc
        return carry

    lax.fori_loop(0, _BPW, body, 0)

    lane = lax.iota(jnp.int32, _LANES)
    for g in range(_BPW // _LANES):
        rows = lane + (g * _LANES)
        tot = jnp.zeros((_LANES,), jnp.float32)
        for j in range(_LANES):
            tot = tot + plsc.load_gather(pv, [rows, jnp.full((_LANES,), j,
                                                             jnp.int32)])
        ov[pl.ds(g * _LANES, _LANES)] = _GAMMA - tot
    pltpu.sync_copy(ov, out_hbm.at[pl.ds(base, _BPW)])


@functools.partial(
    pl.kernel,
    mesh=plsc.VectorSubcoreMesh(core_axis_name="c", subcore_axis_name="s"),
    compiler_params=pltpu.CompilerParams(needs_layout_passes=False),
    out_type=jax.ShapeDtypeStruct((_B,), jnp.float32),
    scratch_types=[
        pltpu.VMEM((_BPW,), jnp.int32),
        pltpu.VMEM((_BPW,), jnp.int32),
        pltpu.VMEM((_BPW,), jnp.int32),
        pltpu.VMEM((_BPW, 2 * _HID), jnp.float32),
        pltpu.VMEM((_BPW, 2 * _HID), jnp.float32),
        pltpu.VMEM((_BPW, 2 * _HID), jnp.float32),
        pltpu.VMEM((_BPW, 17), jnp.float32),
        pltpu.VMEM((_BPW,), jnp.float32),
        pltpu.SemaphoreType.DMA,
        pltpu.SemaphoreType.DMA,
        pltpu.SemaphoreType.DMA,
    ],
)
def _sc_kernel(ent_hbm, cs_hbm, hidx_hbm, ridx_hbm, tidx_hbm, out_hbm, *rest):
    _sc_score(ent_hbm, cs_hbm, hidx_hbm, ridx_hbm, tidx_hbm, out_hbm, *rest)


def kernel(sample, ent_emb, rel_emb):
    sample = sample.astype(jnp.int32)
    h_idx = sample[:, 0]
    r_idx = sample[:, 1]
    t_idx = sample[:, 2]
    cs = _make_cs_table(rel_emb)
    out = _sc_kernel(ent_emb, cs, h_idx, r_idx, t_idx)
    return out.reshape(_B, 1)
